# baseline (speedup 1.0000x reference)
import functools

import jax
import jax.numpy as jnp
from jax.experimental import pallas as pl
from jax.experimental.pallas import tpu as pltpu

_BN = 2048


def _stats_body(t_ref, q_ref, w_ref, train_ref, m_ref, s_ref, zq_ref, *,
                n_total):
    nb = pl.program_id(0)

    @pl.when(nb == 0)
    def _init():
        zq_ref[...] = jnp.dot(q_ref[...], w_ref[...],
                              preferred_element_type=jnp.float32)
        m_ref[...] = jnp.full_like(m_ref, -1e30)
        s_ref[...] = jnp.zeros_like(s_ref)

    zt = jnp.dot(train_ref[...], w_ref[...],
                 preferred_element_type=jnp.float32)
    sim = jnp.dot(zq_ref[...], zt.T, preferred_element_type=jnp.float32)
    logits = sim / t_ref[0]
    col = nb * _BN + jax.lax.broadcasted_iota(jnp.int32, logits.shape, 1)
    logits = jnp.where(col < n_total, logits, -1e30)

    m_old = m_ref[...]
    m_new = jnp.maximum(m_old, jnp.max(logits, axis=1, keepdims=True))
    s_ref[...] = (s_ref[...] * jnp.exp(m_old - m_new)
                  + jnp.sum(jnp.exp(logits - m_new), axis=1, keepdims=True))
    m_ref[...] = m_new


def _combine_body(t_ref, q_ref, w_ref, train_ref, act_ref, m_ref, s_ref,
                  wout_ref, pred_ref, zq_ref, *, n_total):
    nb = pl.program_id(0)

    @pl.when(nb == 0)
    def _init():
        zq_ref[...] = jnp.dot(q_ref[...], w_ref[...],
                              preferred_element_type=jnp.float32)
        pred_ref[...] = jnp.zeros_like(pred_ref)

    zt = jnp.dot(train_ref[...], w_ref[...],
                 preferred_element_type=jnp.float32)
    sim = jnp.dot(zq_ref[...], zt.T, preferred_element_type=jnp.float32)
    logits = sim / t_ref[0]
    col = nb * _BN + jax.lax.broadcasted_iota(jnp.int32, logits.shape, 1)
    w = jnp.exp(logits - m_ref[...]) / s_ref[...]
    w = jnp.where(col < n_total, w, 0.0)
    wout_ref[...] = w

    row = jax.lax.broadcasted_iota(jnp.int32, act_ref.shape, 0) + nb * _BN
    act = jnp.where(row < n_total, act_ref[...], 0.0)
    pred_ref[...] += jnp.dot(w, act, preferred_element_type=jnp.float32)


def kernel(query_obs, train_obs, train_actions, W_enc, log_temperature):
    B, d = query_obs.shape
    N = train_obs.shape[0]
    H, A = train_actions.shape[1], train_actions.shape[2]
    HA = H * A
    nb_total = pl.cdiv(N, _BN)

    temp = jnp.exp(log_temperature).reshape(1)
    act_flat = train_actions.reshape(N, HA)

    scalar_spec = pl.BlockSpec(memory_space=pltpu.SMEM)

    m, s = pl.pallas_call(
        functools.partial(_stats_body, n_total=N),
        grid=(nb_total,),
        in_specs=[
            scalar_spec,
            pl.BlockSpec((B, d), lambda nb: (0, 0)),
            pl.BlockSpec((d, d), lambda nb: (0, 0)),
            pl.BlockSpec((_BN, d), lambda nb: (nb, 0)),
        ],
        out_specs=[
            pl.BlockSpec((B, 1), lambda nb: (0, 0)),
            pl.BlockSpec((B, 1), lambda nb: (0, 0)),
        ],
        out_shape=[
            jax.ShapeDtypeStruct((B, 1), jnp.float32),
            jax.ShapeDtypeStruct((B, 1), jnp.float32),
        ],
        scratch_shapes=[pltpu.VMEM((B, d), jnp.float32)],
        compiler_params=pltpu.CompilerParams(
            dimension_semantics=("arbitrary",),
        ),
    )(temp, query_obs, W_enc, train_obs)

    weights, pred = pl.pallas_call(
        functools.partial(_combine_body, n_total=N),
        grid=(nb_total,),
        in_specs=[
            scalar_spec,
            pl.BlockSpec((B, d), lambda nb: (0, 0)),
            pl.BlockSpec((d, d), lambda nb: (0, 0)),
            pl.BlockSpec((_BN, d), lambda nb: (nb, 0)),
            pl.BlockSpec((_BN, HA), lambda nb: (nb, 0)),
            pl.BlockSpec((B, 1), lambda nb: (0, 0)),
            pl.BlockSpec((B, 1), lambda nb: (0, 0)),
        ],
        out_specs=[
            pl.BlockSpec((B, _BN), lambda nb: (0, nb)),
            pl.BlockSpec((B, HA), lambda nb: (0, 0)),
        ],
        out_shape=[
            jax.ShapeDtypeStruct((B, N), jnp.float32),
            jax.ShapeDtypeStruct((B, HA), jnp.float32),
        ],
        scratch_shapes=[pltpu.VMEM((B, d), jnp.float32)],
        compiler_params=pltpu.CompilerParams(
            dimension_semantics=("arbitrary",),
        ),
    )(temp, query_obs, W_enc, train_obs, act_flat, m, s)

    return (pred.reshape(B, H, A), weights)
